# baseline (speedup 1.0000x reference)
import functools

import jax
import jax.numpy as jnp
from jax import lax
from jax.experimental import pallas as pl
from jax.experimental.pallas import tpu as pltpu
from jax.experimental.pallas import tpu_sc as plsc

_B = 16384
_D = 64
_H = 256
_V = 100001
_NC = 2
_NS = 16
_NW = _NC * _NS
_BPW = _B // _NW
_CHUNK = 128
_NCH = _BPW // _CHUNK
_BLK = 2048


def _mesh():
    return plsc.VectorSubcoreMesh(core_axis_name="c", subcore_axis_name="s")


_FBLK = 2048
_FGRID = (_V + _FBLK - 1) // _FBLK
_FH = _FBLK // 2
_LINR = _FGRID * _FH


def _fmt_body(x_ref, o_ref):
    xt = x_ref[...].T
    o_ref[...] = jnp.concatenate([xt[0:_FH], xt[_FH:_FBLK]], axis=1)


def _tc_format(tableT):
    return pl.pallas_call(
        _fmt_body,
        grid=(_FGRID,),
        in_specs=[pl.BlockSpec((_D, _FBLK), lambda i: (0, i))],
        out_specs=pl.BlockSpec((_FH, 128), lambda i: (i, 0)),
        out_shape=jax.ShapeDtypeStruct((_LINR, 128), jnp.float32),
    )(tableT)


def _sc_gather(lin, labels):
    @functools.partial(
        pl.kernel,
        mesh=_mesh(),
        out_type=jax.ShapeDtypeStruct((_B, 128), jnp.float32),
        scratch_types=[
            pltpu.VMEM((_BPW,), jnp.int32),
            pltpu.VMEM((_BPW,), jnp.int32),
            pltpu.VMEM((_BPW, 128), jnp.float32),
            pltpu.SemaphoreType.DMA,
        ],
        compiler_params=pltpu.CompilerParams(needs_layout_passes=False),
    )
    def k(lin_hbm, lab_hbm, out_hbm, lab_v, pidx_v, rows_v, sem):
        w = lax.axis_index("s") * _NC + lax.axis_index("c")
        base = w * _BPW
        pltpu.sync_copy(lab_hbm.at[pl.ds(base, _BPW)], lab_v)
        for t in range(_BPW // 16):
            lv = lab_v[pl.ds(16 * t, 16)]
            pidx_v[pl.ds(16 * t, 16)] = ((lv >> 11) << 10) + (lv & 1023)
        copies = []
        for j in range(_NCH):
            copies.append(
                pltpu.async_copy(
                    lin_hbm.at[pidx_v.at[pl.ds(j * _CHUNK, _CHUNK)]],
                    rows_v.at[pl.ds(j * _CHUNK, _CHUNK)],
                    sem,
                )
            )
        for c in copies:
            c.wait()
        pltpu.sync_copy(rows_v, out_hbm.at[pl.ds(base, _BPW)])

    return k(lin, labels)


def _mlp_body(x_ref, lab_ref, w1_ref, b1_ref, w2_ref, b2_ref, o_ref):
    x = x_ref[...]
    half = (lab_ref[...].reshape(_BLK, 1) >> 10) & 1
    e = jnp.where(half == 1, x[:, _D:2 * _D], x[:, 0:_D])
    h = jnp.dot(e, w1_ref[...], preferred_element_type=jnp.float32)
    h = h + b1_ref[...]
    h = h * jax.nn.sigmoid(h)
    o = jnp.dot(h, w2_ref[...], preferred_element_type=jnp.float32)
    o_ref[...] = (o + b2_ref[...]).T


def _tc_mlp(emb128, labels3, W1, b1, W2, b2):
    grid = (_B // _BLK,)
    return pl.pallas_call(
        _mlp_body,
        grid=grid,
        in_specs=[
            pl.BlockSpec((_BLK, 128), lambda i: (i, 0)),
            pl.BlockSpec((1, 1, _BLK), lambda i: (i, 0, 0)),
            pl.BlockSpec((_D, _H), lambda i: (0, 0)),
            pl.BlockSpec((1, _H), lambda i: (0, 0)),
            pl.BlockSpec((_H, _D), lambda i: (0, 0)),
            pl.BlockSpec((1, _D), lambda i: (0, 0)),
        ],
        out_specs=pl.BlockSpec((_D, _BLK), lambda i: (0, i)),
        out_shape=jax.ShapeDtypeStruct((_D, _B), jnp.float32),
    )(emb128, labels3, W1, b1, W2, b2)


def kernel(class_labels, table, W1, b1, W2, b2):
    labels = class_labels.astype(jnp.int32)
    lin = _tc_format(table.T)
    emb128 = _sc_gather(lin, labels)
    labels3 = labels.reshape(_B // _BLK, 1, _BLK)
    outT = _tc_mlp(emb128, labels3, W1, b1.reshape(1, _H), W2, b2.reshape(1, _D))
    return outT.T


# speedup vs baseline: 1.1638x; 1.1638x over previous
import functools

import jax
import jax.numpy as jnp
from jax import lax
from jax.experimental import pallas as pl
from jax.experimental.pallas import tpu as pltpu
from jax.experimental.pallas import tpu_sc as plsc

_B = 16384
_D = 64
_H = 256
_V = 100001
_NC = 2
_NS = 16
_NW = _NC * _NS
_BPW = _B // _NW
_CHUNK = 128
_NCH = _BPW // _CHUNK
_BLK = 4096


def _mesh():
    return plsc.VectorSubcoreMesh(core_axis_name="c", subcore_axis_name="s")


_FBLK = 4096
_FGRID = (_V + _FBLK - 1) // _FBLK
_FH = _FBLK // 2
_FSH = _FBLK.bit_length() - 1
_LINR = _FGRID * _FH


def _fmt_body(x_ref, o_ref):
    xt = x_ref[...].T
    o_ref[...] = jnp.concatenate([xt[0:_FH], xt[_FH:_FBLK]], axis=1)


def _tc_format(tableT):
    return pl.pallas_call(
        _fmt_body,
        grid=(_FGRID,),
        in_specs=[pl.BlockSpec((_D, _FBLK), lambda i: (0, i))],
        out_specs=pl.BlockSpec((_FH, 128), lambda i: (i, 0)),
        out_shape=jax.ShapeDtypeStruct((_LINR, 128), jnp.float32),
    )(tableT)


def _sc_gather(lin, labels):
    @functools.partial(
        pl.kernel,
        mesh=_mesh(),
        out_type=jax.ShapeDtypeStruct((_B, 128), jnp.float32),
        scratch_types=[
            pltpu.VMEM((_BPW,), jnp.int32),
            pltpu.VMEM((_BPW,), jnp.int32),
            pltpu.VMEM((_BPW, 128), jnp.float32),
            pltpu.SemaphoreType.DMA,
        ],
        compiler_params=pltpu.CompilerParams(needs_layout_passes=False),
    )
    def k(lin_hbm, lab_hbm, out_hbm, lab_v, pidx_v, rows_v, sem):
        w = lax.axis_index("s") * _NC + lax.axis_index("c")
        base = w * _BPW
        pltpu.sync_copy(lab_hbm.at[pl.ds(base, _BPW)], lab_v)
        for t in range(_BPW // 16):
            lv = lab_v[pl.ds(16 * t, 16)]
            pidx_v[pl.ds(16 * t, 16)] = ((lv >> _FSH) << (_FSH - 1)) + (
                lv & (_FH - 1)
            )
        copies = []
        for j in range(_NCH):
            copies.append(
                pltpu.async_copy(
                    lin_hbm.at[pidx_v.at[pl.ds(j * _CHUNK, _CHUNK)]],
                    rows_v.at[pl.ds(j * _CHUNK, _CHUNK)],
                    sem,
                )
            )
        for c in copies:
            c.wait()
        pltpu.sync_copy(rows_v, out_hbm.at[pl.ds(base, _BPW)])

    return k(lin, labels)


def _mlp_body(x_ref, lab_ref, w1_ref, b1_ref, w2_ref, b2_ref, o_ref):
    x = x_ref[...]
    half = (lab_ref[...].reshape(_BLK, 1) >> (_FSH - 1)) & 1
    e = jnp.where(half == 1, x[:, _D:2 * _D], x[:, 0:_D])
    h = jnp.dot(e, w1_ref[...], preferred_element_type=jnp.float32)
    h = h + b1_ref[...]
    h = h * jax.nn.sigmoid(h)
    o = jnp.dot(h, w2_ref[...], preferred_element_type=jnp.float32)
    o_ref[...] = (o + b2_ref[...]).T


def _tc_mlp(emb128, labels3, W1, b1, W2, b2):
    grid = (_B // _BLK,)
    return pl.pallas_call(
        _mlp_body,
        grid=grid,
        in_specs=[
            pl.BlockSpec((_BLK, 128), lambda i: (i, 0)),
            pl.BlockSpec((1, 1, _BLK), lambda i: (i, 0, 0)),
            pl.BlockSpec((_D, _H), lambda i: (0, 0)),
            pl.BlockSpec((1, _H), lambda i: (0, 0)),
            pl.BlockSpec((_H, _D), lambda i: (0, 0)),
            pl.BlockSpec((1, _D), lambda i: (0, 0)),
        ],
        out_specs=pl.BlockSpec((_D, _BLK), lambda i: (0, i)),
        out_shape=jax.ShapeDtypeStruct((_D, _B), jnp.float32),
    )(emb128, labels3, W1, b1, W2, b2)


def kernel(class_labels, table, W1, b1, W2, b2):
    labels = class_labels.astype(jnp.int32)
    lin = _tc_format(table.T)
    emb128 = _sc_gather(lin, labels)
    labels3 = labels.reshape(_B // _BLK, 1, _BLK)
    outT = _tc_mlp(emb128, labels3, W1, b1.reshape(1, _H), W2, b2.reshape(1, _D))
    return outT.T


# speedup vs baseline: 1.2446x; 1.0694x over previous
import functools

import jax
import jax.numpy as jnp
from jax import lax
from jax.experimental import pallas as pl
from jax.experimental.pallas import tpu as pltpu
from jax.experimental.pallas import tpu_sc as plsc

_B = 16384
_D = 64
_H = 256
_V = 100001
_NC = 2
_NS = 16
_NW = _NC * _NS
_BPW = _B // _NW
_CHUNK = 128
_NCH = _BPW // _CHUNK
_BLK = 8192


def _mesh():
    return plsc.VectorSubcoreMesh(core_axis_name="c", subcore_axis_name="s")


_FBLK = 8192
_FGRID = (_V + _FBLK - 1) // _FBLK
_FH = _FBLK // 2
_FSH = _FBLK.bit_length() - 1
_LINR = _FGRID * _FH


def _fmt_body(x_ref, o_ref):
    xt = x_ref[...].T
    o_ref[...] = jnp.concatenate([xt[0:_FH], xt[_FH:_FBLK]], axis=1)


def _tc_format(tableT):
    return pl.pallas_call(
        _fmt_body,
        grid=(_FGRID,),
        in_specs=[pl.BlockSpec((_D, _FBLK), lambda i: (0, i))],
        out_specs=pl.BlockSpec((_FH, 128), lambda i: (i, 0)),
        out_shape=jax.ShapeDtypeStruct((_LINR, 128), jnp.float32),
    )(tableT)


def _sc_gather(lin, labels):
    @functools.partial(
        pl.kernel,
        mesh=_mesh(),
        out_type=jax.ShapeDtypeStruct((_B, 128), jnp.float32),
        scratch_types=[
            pltpu.VMEM((_BPW,), jnp.int32),
            pltpu.VMEM((_BPW,), jnp.int32),
            pltpu.VMEM((_BPW, 128), jnp.float32),
            pltpu.SemaphoreType.DMA,
        ],
        compiler_params=pltpu.CompilerParams(needs_layout_passes=False),
    )
    def k(lin_hbm, lab_hbm, out_hbm, lab_v, pidx_v, rows_v, sem):
        w = lax.axis_index("s") * _NC + lax.axis_index("c")
        base = w * _BPW
        pltpu.sync_copy(lab_hbm.at[pl.ds(base, _BPW)], lab_v)
        for t in range(_BPW // 16):
            lv = lab_v[pl.ds(16 * t, 16)]
            pidx_v[pl.ds(16 * t, 16)] = ((lv >> _FSH) << (_FSH - 1)) + (
                lv & (_FH - 1)
            )
        copies = []
        for j in range(_NCH):
            copies.append(
                pltpu.async_copy(
                    lin_hbm.at[pidx_v.at[pl.ds(j * _CHUNK, _CHUNK)]],
                    rows_v.at[pl.ds(j * _CHUNK, _CHUNK)],
                    sem,
                )
            )
        for c in copies:
            c.wait()
        pltpu.sync_copy(rows_v, out_hbm.at[pl.ds(base, _BPW)])

    return k(lin, labels)


def _mlp_body(x_ref, lab_ref, w1_ref, b1_ref, w2_ref, b2_ref, o_ref):
    x = x_ref[...]
    half = (lab_ref[...].reshape(_BLK, 1) >> (_FSH - 1)) & 1
    e = jnp.where(half == 1, x[:, _D:2 * _D], x[:, 0:_D])
    h = jnp.dot(e, w1_ref[...], preferred_element_type=jnp.float32)
    h = h + b1_ref[...]
    h = h * jax.nn.sigmoid(h)
    o = jnp.dot(h, w2_ref[...], preferred_element_type=jnp.float32)
    o_ref[...] = (o + b2_ref[...]).T


def _tc_mlp(emb128, labels3, W1, b1, W2, b2):
    grid = (_B // _BLK,)
    return pl.pallas_call(
        _mlp_body,
        grid=grid,
        in_specs=[
            pl.BlockSpec((_BLK, 128), lambda i: (i, 0)),
            pl.BlockSpec((1, 1, _BLK), lambda i: (i, 0, 0)),
            pl.BlockSpec((_D, _H), lambda i: (0, 0)),
            pl.BlockSpec((1, _H), lambda i: (0, 0)),
            pl.BlockSpec((_H, _D), lambda i: (0, 0)),
            pl.BlockSpec((1, _D), lambda i: (0, 0)),
        ],
        out_specs=pl.BlockSpec((_D, _BLK), lambda i: (0, i)),
        out_shape=jax.ShapeDtypeStruct((_D, _B), jnp.float32),
    )(emb128, labels3, W1, b1, W2, b2)


def kernel(class_labels, table, W1, b1, W2, b2):
    labels = class_labels.astype(jnp.int32)
    lin = _tc_format(table.T)
    emb128 = _sc_gather(lin, labels)
    labels3 = labels.reshape(_B // _BLK, 1, _BLK)
    outT = _tc_mlp(emb128, labels3, W1, b1.reshape(1, _H), W2, b2.reshape(1, _D))
    return outT.T


# speedup vs baseline: 1.2508x; 1.0050x over previous
import functools

import jax
import jax.numpy as jnp
from jax import lax
from jax.experimental import pallas as pl
from jax.experimental.pallas import tpu as pltpu
from jax.experimental.pallas import tpu_sc as plsc

_B = 16384
_D = 64
_H = 256
_V = 100001
_NC = 2
_NS = 16
_NW = _NC * _NS
_BPW = _B // _NW
_CHUNK = 128
_NCH = _BPW // _CHUNK
_BLK = 8192


def _mesh():
    return plsc.VectorSubcoreMesh(core_axis_name="c", subcore_axis_name="s")


_FBLK = 16384
_FGRID = (_V + _FBLK - 1) // _FBLK
_FH = _FBLK // 2
_FSH = _FBLK.bit_length() - 1
_LINR = _FGRID * _FH


def _fmt_body(x_ref, o_ref):
    xt = x_ref[...].T
    o_ref[...] = jnp.concatenate([xt[0:_FH], xt[_FH:_FBLK]], axis=1)


def _tc_format(tableT):
    return pl.pallas_call(
        _fmt_body,
        grid=(_FGRID,),
        in_specs=[pl.BlockSpec((_D, _FBLK), lambda i: (0, i))],
        out_specs=pl.BlockSpec((_FH, 128), lambda i: (i, 0)),
        out_shape=jax.ShapeDtypeStruct((_LINR, 128), jnp.float32),
    )(tableT)


def _sc_gather(lin, labels):
    @functools.partial(
        pl.kernel,
        mesh=_mesh(),
        out_type=jax.ShapeDtypeStruct((_B, 128), jnp.float32),
        scratch_types=[
            pltpu.VMEM((_BPW,), jnp.int32),
            pltpu.VMEM((_BPW,), jnp.int32),
            pltpu.VMEM((_BPW, 128), jnp.float32),
            pltpu.SemaphoreType.DMA,
        ],
        compiler_params=pltpu.CompilerParams(needs_layout_passes=False),
    )
    def k(lin_hbm, lab_hbm, out_hbm, lab_v, pidx_v, rows_v, sem):
        w = lax.axis_index("s") * _NC + lax.axis_index("c")
        base = w * _BPW
        pltpu.sync_copy(lab_hbm.at[pl.ds(base, _BPW)], lab_v)
        for t in range(_BPW // 16):
            lv = lab_v[pl.ds(16 * t, 16)]
            pidx_v[pl.ds(16 * t, 16)] = ((lv >> _FSH) << (_FSH - 1)) + (
                lv & (_FH - 1)
            )
        copies = []
        for j in range(_NCH):
            copies.append(
                pltpu.async_copy(
                    lin_hbm.at[pidx_v.at[pl.ds(j * _CHUNK, _CHUNK)]],
                    rows_v.at[pl.ds(j * _CHUNK, _CHUNK)],
                    sem,
                )
            )
        for c in copies:
            c.wait()
        pltpu.sync_copy(rows_v, out_hbm.at[pl.ds(base, _BPW)])

    return k(lin, labels)


def _mlp_body(x_ref, lab_ref, w1_ref, b1_ref, w2_ref, b2_ref, o_ref):
    x = x_ref[...]
    half = (lab_ref[...].reshape(_BLK, 1) >> (_FSH - 1)) & 1
    e = jnp.where(half == 1, x[:, _D:2 * _D], x[:, 0:_D])
    h = jnp.dot(e, w1_ref[...], preferred_element_type=jnp.float32)
    h = h + b1_ref[...]
    h = h * jax.nn.sigmoid(h)
    o = jnp.dot(h, w2_ref[...], preferred_element_type=jnp.float32)
    o_ref[...] = (o + b2_ref[...]).T


def _tc_mlp(emb128, labels3, W1, b1, W2, b2):
    grid = (_B // _BLK,)
    return pl.pallas_call(
        _mlp_body,
        grid=grid,
        in_specs=[
            pl.BlockSpec((_BLK, 128), lambda i: (i, 0)),
            pl.BlockSpec((1, 1, _BLK), lambda i: (i, 0, 0)),
            pl.BlockSpec((_D, _H), lambda i: (0, 0)),
            pl.BlockSpec((1, _H), lambda i: (0, 0)),
            pl.BlockSpec((_H, _D), lambda i: (0, 0)),
            pl.BlockSpec((1, _D), lambda i: (0, 0)),
        ],
        out_specs=pl.BlockSpec((_D, _BLK), lambda i: (0, i)),
        out_shape=jax.ShapeDtypeStruct((_D, _B), jnp.float32),
    )(emb128, labels3, W1, b1, W2, b2)


def kernel(class_labels, table, W1, b1, W2, b2):
    labels = class_labels.astype(jnp.int32)
    lin = _tc_format(table.T)
    emb128 = _sc_gather(lin, labels)
    labels3 = labels.reshape(_B // _BLK, 1, _BLK)
    outT = _tc_mlp(emb128, labels3, W1, b1.reshape(1, _H), W2, b2.reshape(1, _D))
    return outT.T


# speedup vs baseline: 1.3031x; 1.0418x over previous
import functools

import jax
import jax.numpy as jnp
from jax import lax
from jax.experimental import pallas as pl
from jax.experimental.pallas import tpu as pltpu
from jax.experimental.pallas import tpu_sc as plsc

_B = 16384
_D = 64
_H = 256
_V = 100001
_NC = 2
_NS = 16
_NW = _NC * _NS
_BPW = _B // _NW
_CHUNK = 128
_NCH = _BPW // _CHUNK
_BLK = 8192


def _mesh():
    return plsc.VectorSubcoreMesh(core_axis_name="c", subcore_axis_name="s")


_FBLK = 16384
_FGRID = (_V + _FBLK - 1) // _FBLK
_FH = _FBLK // 2
_FSH = _FBLK.bit_length() - 1
_LINR = _FGRID * _FH


def _fmt_body(x_ref, o_ref):
    xt = x_ref[...].T
    o_ref[...] = jnp.concatenate([xt[0:_FH], xt[_FH:_FBLK]], axis=1)


def _tc_format(tableT):
    return pl.pallas_call(
        _fmt_body,
        grid=(_FGRID,),
        in_specs=[pl.BlockSpec((_D, _FBLK), lambda i: (0, i))],
        out_specs=pl.BlockSpec((_FH, 128), lambda i: (i, 0)),
        out_shape=jax.ShapeDtypeStruct((_LINR, 128), jnp.float32),
    )(tableT)


def _sc_gather(lin, labels):
    @functools.partial(
        pl.kernel,
        mesh=_mesh(),
        out_type=jax.ShapeDtypeStruct((_B, 128), jnp.float32),
        scratch_types=[
            pltpu.VMEM((_BPW,), jnp.int32),
            pltpu.VMEM((_BPW,), jnp.int32),
            pltpu.VMEM((_BPW, 128), jnp.float32),
            pltpu.SemaphoreType.DMA,
        ],
        compiler_params=pltpu.CompilerParams(needs_layout_passes=False),
    )
    def k(lin_hbm, lab_hbm, out_hbm, lab_v, pidx_v, rows_v, sem):
        w = lax.axis_index("s") * _NC + lax.axis_index("c")
        base = w * _BPW
        pltpu.sync_copy(lab_hbm.at[pl.ds(base, _BPW)], lab_v)
        for t in range(_BPW // 16):
            lv = lab_v[pl.ds(16 * t, 16)]
            pidx_v[pl.ds(16 * t, 16)] = ((lv >> _FSH) << (_FSH - 1)) + (
                lv & (_FH - 1)
            )
        copies = []
        for j in range(_NCH):
            copies.append(
                pltpu.async_copy(
                    lin_hbm.at[pidx_v.at[pl.ds(j * _CHUNK, _CHUNK)]],
                    rows_v.at[pl.ds(j * _CHUNK, _CHUNK)],
                    sem,
                )
            )
        for c in copies:
            c.wait()
        pltpu.sync_copy(rows_v, out_hbm.at[pl.ds(base, _BPW)])

    return k(lin, labels)


def _mlp_body(x_ref, lab_ref, w1_ref, b1_ref, w2t_ref, b2t_ref, o_ref):
    x = x_ref[...]
    half = (lab_ref[...].reshape(_BLK, 1) >> (_FSH - 1)) & 1
    e = jnp.where(half == 1, x[:, _D:2 * _D], x[:, 0:_D])
    h = jnp.dot(e, w1_ref[...], preferred_element_type=jnp.float32)
    h = h + b1_ref[...]
    h = h * jax.nn.sigmoid(h)
    ot = lax.dot_general(
        w2t_ref[...], h, (((1,), (1,)), ((), ())),
        preferred_element_type=jnp.float32,
    )
    o_ref[...] = ot + b2t_ref[...]


def _tc_mlp(emb128, labels3, W1, b1, W2, b2):
    grid = (_B // _BLK,)
    return pl.pallas_call(
        _mlp_body,
        grid=grid,
        in_specs=[
            pl.BlockSpec((_BLK, 128), lambda i: (i, 0)),
            pl.BlockSpec((1, 1, _BLK), lambda i: (i, 0, 0)),
            pl.BlockSpec((_D, _H), lambda i: (0, 0)),
            pl.BlockSpec((1, _H), lambda i: (0, 0)),
            pl.BlockSpec((_D, _H), lambda i: (0, 0)),
            pl.BlockSpec((_D, 1), lambda i: (0, 0)),
        ],
        out_specs=pl.BlockSpec((_D, _BLK), lambda i: (0, i)),
        out_shape=jax.ShapeDtypeStruct((_D, _B), jnp.float32),
    )(emb128, labels3, W1, b1, W2, b2)


def kernel(class_labels, table, W1, b1, W2, b2):
    labels = class_labels.astype(jnp.int32)
    lin = _tc_format(table.T)
    emb128 = _sc_gather(lin, labels)
    labels3 = labels.reshape(_B // _BLK, 1, _BLK)
    outT = _tc_mlp(
        emb128, labels3, W1, b1.reshape(1, _H), W2.T, b2.reshape(_D, 1)
    )
    return outT.T
